# Initial kernel scaffold; baseline (speedup 1.0000x reference)
import jax
import jax.numpy as jnp
from jax.experimental import pallas as pl


def kernel(preds, targets):
    raise NotImplementedError("write your pallas kernel here")


# speedup vs baseline: 2.4922x; 2.4922x over previous
import functools
import math

import jax
import jax.numpy as jnp
from jax import lax
from jax.experimental import pallas as pl

NC = 80
IOU_THR = 0.45
HYP_CLS = 0.5
HYP_BOX = 7.5
EPS = 1e-7
BN = 512


def _dense_body(n_real, m_real, pt_ref, tg_ref, acc_ref, mat_ref):
    b = pl.program_id(0)
    px1 = pt_ref[0:1, :]
    py1 = pt_ref[1:2, :]
    px2 = pt_ref[2:3, :]
    py2 = pt_ref[3:4, :]
    ps = pt_ref[4:5, :]
    pc = pt_ref[5:6, :]
    tx1 = tg_ref[:, 0:1]
    ty1 = tg_ref[:, 1:2]
    tx2 = tg_ref[:, 2:3]
    ty2 = tg_ref[:, 3:4]
    tc = tg_ref[:, 4:5]

    ap = (px2 - px1) * (py2 - py1)
    at = (tx2 - tx1) * (ty2 - ty1)
    iw = jnp.maximum(jnp.minimum(px2, tx2) - jnp.maximum(px1, tx1), 0.0)
    ih = jnp.maximum(jnp.minimum(py2, ty2) - jnp.maximum(py1, ty1), 0.0)
    inter = iw * ih
    iou = inter / (ap + at - inter + EPS)

    mx = jnp.max(iou, axis=0, keepdims=True)
    jid = lax.broadcasted_iota(jnp.int32, iou.shape, 0)
    cand = jnp.where(iou == mx, jid, jnp.int32(2 ** 30))
    idx = jnp.min(cand, axis=0, keepdims=True)
    sel = jid == idx

    bx1 = jnp.sum(jnp.where(sel, tx1, 0.0), axis=0, keepdims=True)
    by1 = jnp.sum(jnp.where(sel, ty1, 0.0), axis=0, keepdims=True)
    bx2 = jnp.sum(jnp.where(sel, tx2, 0.0), axis=0, keepdims=True)
    by2 = jnp.sum(jnp.where(sel, ty2, 0.0), axis=0, keepdims=True)
    bc = jnp.sum(jnp.where(sel, tc, 0.0), axis=0, keepdims=True)

    col = b * BN + lax.broadcasted_iota(jnp.int32, (1, BN), 1)
    valid = col < n_real
    keep = (mx > IOU_THR) & valid
    kf = keep.astype(jnp.float32)

    mm = jnp.where(sel & keep, 1.0, 0.0)
    mm128 = mm[:, 0:128]
    for i in range(1, BN // 128):
        mm128 = jnp.maximum(mm128, mm[:, i * 128:(i + 1) * 128])

    s_safe = jnp.where(keep, ps, 0.5)
    cls_match = pc == bc
    bce = kf * jnp.where(cls_match, -jnp.log(s_safe),
                         100.0 - jnp.maximum(jnp.log1p(-s_safe), -100.0))

    unm_mask = valid & ~keep
    u = jnp.where(unm_mask, jnp.log(jnp.where(unm_mask, ps, 0.5)), 0.0)

    w1 = px2 - px1
    h1 = py2 - py1 + EPS
    w2 = bx2 - bx1
    h2 = by2 - by1 + EPS
    iw2 = jnp.maximum(jnp.minimum(px2, bx2) - jnp.maximum(px1, bx1), 0.0)
    ih2 = jnp.maximum(jnp.minimum(py2, by2) - jnp.maximum(py1, by1), 0.0)
    inter2 = iw2 * ih2
    union = w1 * h1 + w2 * h2 - inter2 + EPS
    iou2 = inter2 / union
    cw = jnp.maximum(px2, bx2) - jnp.minimum(px1, bx1)
    ch = jnp.maximum(py2, by2) - jnp.minimum(py1, by1)
    c2 = cw * cw + ch * ch + EPS
    rho2 = ((bx1 + bx2 - px1 - px2) ** 2 + (by1 + by2 - py1 - py2) ** 2) / 4.0
    v = (4.0 / math.pi ** 2) * (jnp.arctan2(w2, h2) - jnp.arctan2(w1, h1)) ** 2
    alpha = v / (v - iou2 + (1.0 + EPS))
    ciou = iou2 - (rho2 / c2 + v * alpha)
    bb = kf * (1.0 - ciou)

    def fold(x):
        r = x[:, 0:128]
        for i in range(1, BN // 128):
            r = r + x[:, i * 128:(i + 1) * 128]
        return r

    part = jnp.concatenate(
        [fold(kf), fold(bce), fold(u), fold(bb),
         jnp.zeros((4, 128), jnp.float32)], axis=0)

    @pl.when(b == 0)
    def _():
        acc_ref[...] = jnp.zeros_like(acc_ref)
        mat_ref[...] = jnp.zeros_like(mat_ref)

    acc_ref[...] += part
    mat_ref[...] = jnp.maximum(mat_ref[...], mm128)


def _final_body(m_real, acc_ref, mat_ref, out_ref):
    acc = acc_ref[...]
    kf_t = jnp.sum(acc[0:1, :])
    bce_t = jnp.sum(acc[1:2, :])
    u_t = jnp.sum(acc[2:3, :])
    bb_t = jnp.sum(acc[3:4, :])
    mat = mat_ref[...]
    cnt = jnp.sum(jnp.max(mat, axis=1, keepdims=True))
    n_kept = jnp.maximum(kf_t, 1.0)
    cls_loss = bce_t / (n_kept * NC) - u_t + (m_real - cnt)
    total = HYP_CLS * cls_loss + HYP_BOX * bb_t / n_kept
    out_ref[...] = jnp.full((8, 128), total, jnp.float32)


def kernel(preds, targets):
    n, _ = preds.shape
    m, _ = targets.shape
    np_ = ((n + BN - 1) // BN) * BN
    mp = ((m + 7) // 8) * 8

    pt = jnp.zeros((8, np_), jnp.float32).at[:6, :n].set(preds.T)
    tg = jnp.zeros((mp, 5), jnp.float32).at[:m, :].set(targets)

    grid = np_ // BN
    acc, mat = pl.pallas_call(
        functools.partial(_dense_body, n, m),
        grid=(grid,),
        in_specs=[
            pl.BlockSpec((8, BN), lambda b: (0, b)),
            pl.BlockSpec((mp, 5), lambda b: (0, 0)),
        ],
        out_specs=[
            pl.BlockSpec((8, 128), lambda b: (0, 0)),
            pl.BlockSpec((mp, 128), lambda b: (0, 0)),
        ],
        out_shape=[
            jax.ShapeDtypeStruct((8, 128), jnp.float32),
            jax.ShapeDtypeStruct((mp, 128), jnp.float32),
        ],
    )(pt, tg)

    out = pl.pallas_call(
        functools.partial(_final_body, m),
        in_specs=[
            pl.BlockSpec((8, 128), lambda: (0, 0)),
            pl.BlockSpec((mp, 128), lambda: (0, 0)),
        ],
        out_specs=pl.BlockSpec((8, 128), lambda: (0, 0)),
        out_shape=jax.ShapeDtypeStruct((8, 128), jnp.float32),
    )(acc, mat)
    return out[0, 0]


# speedup vs baseline: 3.8187x; 1.5322x over previous
import functools
import math

import jax
import jax.numpy as jnp
from jax import lax
from jax.experimental import pallas as pl

NC = 80
IOU_THR = 0.45
HYP_CLS = 0.5
HYP_BOX = 7.5
EPS = 1e-7
BN = 512


def _dense_body(n_real, pp_ref, tgt_ref, tg_ref, gout_ref, mat_ref):
    b = pl.program_id(0)
    px1 = pp_ref[:, 0:1]
    py1 = pp_ref[:, 1:2]
    px2 = pp_ref[:, 2:3]
    py2 = pp_ref[:, 3:4]
    tx1 = tgt_ref[0:1, :]
    ty1 = tgt_ref[1:2, :]
    tx2 = tgt_ref[2:3, :]
    ty2 = tgt_ref[3:4, :]

    ap = (px2 - px1) * (py2 - py1)
    at = (tx2 - tx1) * (ty2 - ty1)
    iw = jnp.maximum(jnp.minimum(px2, tx2) - jnp.maximum(px1, tx1), 0.0)
    ih = jnp.maximum(jnp.minimum(py2, ty2) - jnp.maximum(py1, ty1), 0.0)
    inter = iw * ih
    iou = inter / (ap + at - inter + EPS)

    mx = jnp.max(iou, axis=1, keepdims=True)
    jid = lax.broadcasted_iota(jnp.int32, iou.shape, 1)
    cand = jnp.where(iou == mx, jid, jnp.int32(2 ** 30))
    idx = jnp.min(cand, axis=1, keepdims=True)
    sel = jid == idx

    row = b * BN + lax.broadcasted_iota(jnp.int32, (BN, 1), 0)
    keep = (mx > IOU_THR) & (row < n_real)
    kf = keep.astype(jnp.float32)

    selk = sel.astype(jnp.float32) * kf
    g = lax.dot_general(selk, tg_ref[...],
                        (((1,), (0,)), ((), ())),
                        preferred_element_type=jnp.float32)
    msum = jnp.sum(selk, axis=0, keepdims=True)

    gout_ref[...] = jnp.concatenate(
        [g[:, :5], mx, kf, jnp.zeros((BN, 1), jnp.float32)], axis=1)

    @pl.when(b == 0)
    def _():
        mat_ref[...] = jnp.zeros_like(mat_ref)

    mat_ref[0:1, :] += msum


def _final_body(n_real, m_real, ppt_ref, gt_ref, mat_ref, out_ref):
    px1 = ppt_ref[0:1, :]
    py1 = ppt_ref[1:2, :]
    px2 = ppt_ref[2:3, :]
    py2 = ppt_ref[3:4, :]
    ps = ppt_ref[4:5, :]
    pc = ppt_ref[5:6, :]
    bx1 = gt_ref[0:1, :]
    by1 = gt_ref[1:2, :]
    bx2 = gt_ref[2:3, :]
    by2 = gt_ref[3:4, :]
    bc = jnp.round(gt_ref[4:5, :])
    kf = gt_ref[6:7, :]
    keep = kf > 0.0
    col = lax.broadcasted_iota(jnp.int32, ps.shape, 1)
    valid = col < n_real

    s_safe = jnp.where(keep, ps, 0.5)
    bce = kf * jnp.where(pc == bc, -jnp.log(s_safe),
                         100.0 - jnp.maximum(jnp.log1p(-s_safe), -100.0))

    unm = valid & ~keep
    u = jnp.where(unm, jnp.log(jnp.where(unm, ps, 0.5)), 0.0)

    w1 = px2 - px1
    h1 = py2 - py1 + EPS
    w2 = bx2 - bx1
    h2 = by2 - by1 + EPS
    iw2 = jnp.maximum(jnp.minimum(px2, bx2) - jnp.maximum(px1, bx1), 0.0)
    ih2 = jnp.maximum(jnp.minimum(py2, by2) - jnp.maximum(py1, by1), 0.0)
    inter2 = iw2 * ih2
    union = w1 * h1 + w2 * h2 - inter2 + EPS
    iou2 = inter2 / union
    cw = jnp.maximum(px2, bx2) - jnp.minimum(px1, bx1)
    ch = jnp.maximum(py2, by2) - jnp.minimum(py1, by1)
    c2 = cw * cw + ch * ch + EPS
    rho2 = ((bx1 + bx2 - px1 - px2) ** 2 + (by1 + by2 - py1 - py2) ** 2) / 4.0
    v = (4.0 / math.pi ** 2) * (jnp.arctan2(w2, h2) - jnp.arctan2(w1, h1)) ** 2
    alpha = v / (v - iou2 + (1.0 + EPS))
    ciou = iou2 - (rho2 / c2 + v * alpha)
    bb = kf * (1.0 - ciou)

    kf_t = jnp.sum(kf)
    bce_t = jnp.sum(bce)
    u_t = jnp.sum(u)
    bb_t = jnp.sum(bb)
    cnt = jnp.sum((mat_ref[0:1, :] > 0.0).astype(jnp.float32))

    n_kept = jnp.maximum(kf_t, 1.0)
    cls_loss = bce_t / (n_kept * NC) - u_t + (m_real - cnt)
    total = HYP_CLS * cls_loss + HYP_BOX * bb_t / n_kept
    out_ref[...] = jnp.full((8, 128), total, jnp.float32)


def kernel(preds, targets):
    n, _ = preds.shape
    m, _ = targets.shape
    np_ = ((n + BN - 1) // BN) * BN
    mp = ((m + 127) // 128) * 128

    pp = jnp.zeros((np_, 8), jnp.float32).at[:n, :6].set(preds)
    ppt = jnp.zeros((8, np_), jnp.float32).at[:6, :n].set(preds.T)
    tgt = jnp.zeros((8, mp), jnp.float32).at[:5, :m].set(targets.T)
    tg = jnp.zeros((mp, 8), jnp.float32).at[:m, :5].set(targets)

    grid = np_ // BN
    gout, mat = pl.pallas_call(
        functools.partial(_dense_body, n),
        grid=(grid,),
        in_specs=[
            pl.BlockSpec((BN, 8), lambda b: (b, 0)),
            pl.BlockSpec((8, mp), lambda b: (0, 0)),
            pl.BlockSpec((mp, 8), lambda b: (0, 0)),
        ],
        out_specs=[
            pl.BlockSpec((BN, 8), lambda b: (b, 0)),
            pl.BlockSpec((8, mp), lambda b: (0, 0)),
        ],
        out_shape=[
            jax.ShapeDtypeStruct((np_, 8), jnp.float32),
            jax.ShapeDtypeStruct((8, mp), jnp.float32),
        ],
    )(pp, tgt, tg)

    out = pl.pallas_call(
        functools.partial(_final_body, n, m),
        in_specs=[
            pl.BlockSpec((8, np_), lambda: (0, 0)),
            pl.BlockSpec((8, np_), lambda: (0, 0)),
            pl.BlockSpec((8, mp), lambda: (0, 0)),
        ],
        out_specs=pl.BlockSpec((8, 128), lambda: (0, 0)),
        out_shape=jax.ShapeDtypeStruct((8, 128), jnp.float32),
    )(ppt, gout.T, mat)
    return out[0, 0]
